# Initial kernel scaffold; baseline (speedup 1.0000x reference)
import functools

import jax
import jax.numpy as jnp
from jax import lax
from jax.experimental import pallas as pl
from jax.experimental.pallas import tpu as pltpu
from jax.experimental.pallas import tpu_sc as plsc

_N = 10000
_E = 160000
_H = 256
_HF = 128
_B = 16
_EPS = 1e-5
_NCONV = 5

_NC = 2
_NS = 16
_NPAD = 10240
_NPT = _NPAD // _NS
_EK = 128
_CONV_CHUNKS = 80
_EPAD = _NS * _EK * _CONV_CHUNKS
_DEG_CHUNKS = _EPAD // (_NC * _NS * _EK)


def _sc_mesh():
    return plsc.VectorSubcoreMesh(core_axis_name="c", subcore_axis_name="s")


@functools.partial(
    pl.kernel,
    out_type=jax.ShapeDtypeStruct((_NC, _NPAD), jnp.float32),
    mesh=_sc_mesh(),
    scratch_types=[
        pltpu.VMEM((_EK,), jnp.int32),
        pltpu.VMEM((_EK,), jnp.float32),
        pltpu.VMEM_SHARED((_NPAD,), jnp.float32),
    ],
)
def _deg_sc(col_hbm, w_hbm, zero_hbm, out_hbm, cidx_v, w_v, acc_sh):
    cid = lax.axis_index("c")
    sid = lax.axis_index("s")
    pltpu.sync_copy(zero_hbm.at[pl.ds(sid * _NPT, _NPT)],
                    acc_sh.at[pl.ds(sid * _NPT, _NPT)])
    plsc.subcore_barrier()
    tile_base = (cid * _NS + sid) * (_EK * _DEG_CHUNKS)

    def chunk(i, _):
        base = pl.multiple_of(tile_base + i * _EK, 8)
        pltpu.sync_copy(col_hbm.at[pl.ds(base, _EK)], cidx_v)
        pltpu.sync_copy(w_hbm.at[pl.ds(base, _EK)], w_v)
        pltpu.sync_copy(w_v, acc_sh.at[cidx_v], add=True)
        return 0

    lax.fori_loop(0, _DEG_CHUNKS, chunk, 0)
    plsc.subcore_barrier()
    pltpu.sync_copy(acc_sh.at[pl.ds(sid * _NPT, _NPT)],
                    out_hbm.at[cid, pl.ds(sid * _NPT, _NPT)])


@functools.partial(
    pl.kernel,
    out_type=jax.ShapeDtypeStruct((_NC, _NPAD, _HF), jnp.float32),
    mesh=_sc_mesh(),
    scratch_types=[
        pltpu.VMEM((_EK,), jnp.int32),
        pltpu.VMEM((_EK,), jnp.int32),
        pltpu.VMEM((_EK,), jnp.float32),
        pltpu.VMEM((_EK, _HF), jnp.float32),
        pltpu.VMEM_SHARED((_NPAD, _HF), jnp.float32),
        pltpu.SemaphoreType.DMA,
    ],
)
def _conv_sc(row_hbm, col_hbm, w_hbm, hwlo_hbm, hwhi_hbm, zero_hbm, out_hbm,
             ridx_v, cidx_v, w_v, rows_v, acc_sh, sem):
    cid = lax.axis_index("c")
    sid = lax.axis_index("s")
    pltpu.sync_copy(zero_hbm.at[pl.ds(sid * _NPT, _NPT)],
                    acc_sh.at[pl.ds(sid * _NPT, _NPT)])
    plsc.subcore_barrier()
    tile_base = sid * (_EK * _CONV_CHUNKS)

    def chunk_body(i, hw_src):
        base = pl.multiple_of(tile_base + i * _EK, 8)
        pltpu.sync_copy(row_hbm.at[pl.ds(base, _EK)], ridx_v)
        pltpu.sync_copy(col_hbm.at[pl.ds(base, _EK)], cidx_v)
        pltpu.sync_copy(w_hbm.at[pl.ds(base, _EK)], w_v)
        pltpu.async_copy(hw_src.at[ridx_v], rows_v, sem).wait()

        def mul_edge(j, _):
            jv = jnp.full((16,), j, jnp.int32)
            wj = plsc.load_gather(w_v, [jv])
            for c in range(_HF // 16):
                sl = (j, pl.ds(c * 16, 16))
                rows_v[sl] = rows_v[sl] * wj
            return 0

        lax.fori_loop(0, _EK, mul_edge, 0)
        pltpu.sync_copy(rows_v, acc_sh.at[cidx_v], add=True)
        return 0

    @pl.when(cid == 0)
    def _():
        lax.fori_loop(0, _CONV_CHUNKS, lambda i, _: chunk_body(i, hwlo_hbm), 0)

    @pl.when(cid == 1)
    def _():
        lax.fori_loop(0, _CONV_CHUNKS, lambda i, _: chunk_body(i, hwhi_hbm), 0)

    plsc.subcore_barrier()
    pltpu.sync_copy(acc_sh.at[pl.ds(sid * _NPT, _NPT)],
                    out_hbm.at[cid, pl.ds(sid * _NPT, _NPT)])


def _mm(a, b):
    return lax.dot_general(a, b, (((1,), (1,)), ((), ())),
                           preferred_element_type=jnp.float32)


def _mmT(a, b):
    return lax.dot_general(a, b, (((0,), (0,)), ((), ())),
                           preferred_element_type=jnp.float32)


def _prep_tc(x_ref, linW_ref, linb_ref, W1_ref, degT_ref,
             dinv_ref, hwlo_ref, hwhi_ref):
    dinv = lax.rsqrt(degT_ref[:_N, 0:1] + degT_ref[:_N, 1:2] + 2.0)
    h0 = jnp.maximum(_mm(x_ref[...], linW_ref[...]) + linb_ref[...], 0.0)
    hw1 = _mm(h0, W1_ref[...]) * dinv
    dinv_ref[...] = dinv
    hwlo_ref[...] = hw1[:, :_HF]
    hwhi_ref[...] = hw1[:, _HF:]


_prep_call = pl.pallas_call(
    _prep_tc,
    out_shape=(
        jax.ShapeDtypeStruct((_N, 1), jnp.float32),
        jax.ShapeDtypeStruct((_N, _HF), jnp.float32),
        jax.ShapeDtypeStruct((_N, _HF), jnp.float32),
    ),
)


def _combine_body(acc2_ref, hwlo_ref, hwhi_ref, dinv_ref, b_ref, g_ref,
                  be_ref):
    dv = dinv_ref[...]
    pre_lo = acc2_ref[0, :_N, :] * dv + 2.0 * dv * hwlo_ref[...]
    pre_hi = acc2_ref[1, :_N, :] * dv + 2.0 * dv * hwhi_ref[...]
    pre = jnp.concatenate([pre_lo, pre_hi], axis=1) + b_ref[...]
    mu = jnp.mean(pre, axis=0, keepdims=True)
    var = jnp.mean((pre - mu) ** 2, axis=0, keepdims=True)
    return (pre - mu) * lax.rsqrt(var + _EPS) * g_ref[...] + be_ref[...]


def _mid_tc(acc2_ref, hwlo_ref, hwhi_ref, dinv_ref, b_ref, g_ref, be_ref,
            Wn_ref, hwlo_o, hwhi_o):
    h = _combine_body(acc2_ref, hwlo_ref, hwhi_ref, dinv_ref, b_ref, g_ref,
                      be_ref)
    hw = _mm(h, Wn_ref[...]) * dinv_ref[...]
    hwlo_o[...] = hw[:, :_HF]
    hwhi_o[...] = hw[:, _HF:]


_mid_call = pl.pallas_call(
    _mid_tc,
    out_shape=(
        jax.ShapeDtypeStruct((_N, _HF), jnp.float32),
        jax.ShapeDtypeStruct((_N, _HF), jnp.float32),
    ),
)


def _fin_tc(acc2_ref, hwlo_ref, hwhi_ref, dinv_ref, b_ref, g_ref, be_ref,
            h_o):
    h = _combine_body(acc2_ref, hwlo_ref, hwhi_ref, dinv_ref, b_ref, g_ref,
                      be_ref)
    h_o[...] = jnp.maximum(h, 0.0)


_fin_call = pl.pallas_call(
    _fin_tc,
    out_shape=jax.ShapeDtypeStruct((_N, _H), jnp.float32),
)


def _s2s_tc(h_ref, batch_ref, Wih_ref, Whh_ref, bih_ref, bhh_ref,
            mW0_ref, mW1_ref, mb0_ref, mb1_ref, o1W_ref, o1b_ref,
            o2W_ref, o2b_ref, out_ref):
    h = h_ref[...]
    onehot = (batch_ref[...] ==
              lax.broadcasted_iota(jnp.int32, (1, _B), 1)).astype(jnp.float32)
    q_star = jnp.zeros((_B, 2 * _H), jnp.float32)
    hh = jnp.zeros((_B, _H), jnp.float32)
    cc = jnp.zeros((_B, _H), jnp.float32)
    for _ in range(3):
        g = (_mm(q_star, Wih_ref[...]) + bih_ref[...] +
             _mm(hh, Whh_ref[...]) + bhh_ref[...])
        ii = g[:, 0 * _H:1 * _H]
        ff = g[:, 1 * _H:2 * _H]
        gg = g[:, 2 * _H:3 * _H]
        oo = g[:, 3 * _H:4 * _H]
        cc = jax.nn.sigmoid(ff) * cc + jax.nn.sigmoid(ii) * jnp.tanh(gg)
        hh = jax.nn.sigmoid(oo) * jnp.tanh(cc)
        q = hh
        hq = _mm(h, q)
        e = jnp.sum(hq * onehot, axis=1, keepdims=True)
        m = jnp.max(jnp.where(onehot > 0.0, hq, -1e30), axis=0,
                    keepdims=True)
        emax = jnp.sum(onehot * m, axis=1, keepdims=True)
        ex = jnp.exp(e - emax)
        den = jnp.sum(onehot * ex, axis=0, keepdims=True)
        dpn = jnp.sum(onehot * den, axis=1, keepdims=True)
        a = ex / (dpn + 1e-16)
        r = _mmT(onehot * a, h)
        q_star = jnp.concatenate([q, r], axis=1)
    z = jnp.maximum(_mm(q_star, mW0_ref[...]) + mb0_ref[...], 0.0)
    z = jnp.maximum(_mm(z, mW1_ref[...]) + mb1_ref[...], 0.0)
    z = jnp.maximum(_mm(z, o1W_ref[...]) + o1b_ref[...], 0.0)
    z = _mm(z, o2W_ref[...]) + o2b_ref[...]
    out_ref[...] = z


_s2s_call = pl.pallas_call(
    _s2s_tc,
    out_shape=jax.ShapeDtypeStruct((_B, 1), jnp.float32),
)


def kernel(x, edge_index, edge_dist, symmetry, global_idx, batch,
           lin_W, lin_b, conv_W, conv_b, bn_gamma, bn_beta,
           W_ih, W_hh, b_ih, b_hh, mlp_W, mlp_b,
           out1_W, out1_b, out2_W, out2_b):
    pad = _EPAD - _E
    rowp = jnp.concatenate([edge_index[0], jnp.zeros((pad,), jnp.int32)])
    colp = jnp.concatenate([edge_index[1], jnp.zeros((pad,), jnp.int32)])
    wp = jnp.concatenate([edge_dist, jnp.zeros((pad,), jnp.float32)])
    zeros1 = jnp.zeros((_NPAD,), jnp.float32)
    zeros2 = jnp.zeros((_NPAD, _HF), jnp.float32)

    deg2 = _deg_sc(colp, wp, zeros1)
    degT = deg2.T
    dinv, hwlo, hwhi = _prep_call(x, lin_W, lin_b.reshape(1, _H),
                                  conv_W[0], degT)
    h_fin = None
    for i in range(_NCONV):
        acc2 = _conv_sc(rowp, colp, wp, hwlo, hwhi, zeros2)
        args = (acc2, hwlo, hwhi, dinv, conv_b[i].reshape(1, _H),
                bn_gamma.reshape(1, _H), bn_beta.reshape(1, _H))
        if i < _NCONV - 1:
            hwlo, hwhi = _mid_call(*args, conv_W[i + 1])
        else:
            h_fin = _fin_call(*args)

    z = _s2s_call(h_fin, batch.reshape(_N, 1),
                  W_ih, W_hh, b_ih.reshape(1, 4 * _H), b_hh.reshape(1, 4 * _H),
                  mlp_W[0], mlp_W[1], mlp_b[0].reshape(1, 2 * _H),
                  mlp_b[1].reshape(1, 2 * _H), out1_W,
                  out1_b.reshape(1, 32), out2_W, out2_b.reshape(1, 1))
    return z.reshape(1, _B)


# speedup vs baseline: 4.4350x; 4.4350x over previous
import functools

import jax
import jax.numpy as jnp
from jax import lax
from jax.experimental import pallas as pl
from jax.experimental.pallas import tpu as pltpu
from jax.experimental.pallas import tpu_sc as plsc

_N = 10000
_E = 160000
_H = 256
_HF = 128
_B = 16
_EPS = 1e-5
_NCONV = 5

_NC = 2
_NS = 16
_NPAD = 10240
_NPT = _NPAD // _NS
_EK = 128
_CONV_CHUNKS = 80
_EPAD = _NS * _EK * _CONV_CHUNKS
_DEG_CHUNKS = _EPAD // (_NC * _NS * _EK)


def _sc_mesh():
    return plsc.VectorSubcoreMesh(core_axis_name="c", subcore_axis_name="s",
                                  num_cores=_NC, num_subcores=_NS)


_SC_CALLS = {}


def _deg_call():
    if "deg" not in _SC_CALLS:
        _SC_CALLS["deg"] = pl.kernel(
            _deg_sc,
            out_type=jax.ShapeDtypeStruct((_NC, _NPAD), jnp.float32),
            mesh=_sc_mesh(),
            scratch_types=[
                pltpu.VMEM((_EK,), jnp.int32),
                pltpu.VMEM((_EK,), jnp.float32),
                pltpu.VMEM_SHARED((_NPAD,), jnp.float32),
            ],
        )
    return _SC_CALLS["deg"]


def _conv_call():
    if "conv" not in _SC_CALLS:
        _SC_CALLS["conv"] = pl.kernel(
            _conv_sc,
            out_type=jax.ShapeDtypeStruct((_NC, _NPAD, _HF), jnp.float32),
            mesh=_sc_mesh(),
            scratch_types=[
                pltpu.VMEM((_EK,), jnp.int32),
                pltpu.VMEM((_EK,), jnp.int32),
                pltpu.VMEM((_EK,), jnp.float32),
                pltpu.VMEM((_EK, _HF), jnp.float32),
                pltpu.VMEM_SHARED((_NPAD, _HF), jnp.float32),
                pltpu.SemaphoreType.DMA,
            ],
        )
    return _SC_CALLS["conv"]


def _deg_sc(col_hbm, w_hbm, zero_hbm, out_hbm, cidx_v, w_v, acc_sh):
    cid = lax.axis_index("c")
    sid = lax.axis_index("s")
    pltpu.sync_copy(zero_hbm.at[pl.ds(sid * _NPT, _NPT)],
                    acc_sh.at[pl.ds(sid * _NPT, _NPT)])
    plsc.subcore_barrier()
    tile_base = (cid * _NS + sid) * (_EK * _DEG_CHUNKS)

    def chunk(i, _):
        base = pl.multiple_of(tile_base + i * _EK, 8)
        pltpu.sync_copy(col_hbm.at[pl.ds(base, _EK)], cidx_v)
        pltpu.sync_copy(w_hbm.at[pl.ds(base, _EK)], w_v)
        pltpu.sync_copy(w_v, acc_sh.at[cidx_v], add=True)
        return 0

    lax.fori_loop(0, _DEG_CHUNKS, chunk, 0)
    plsc.subcore_barrier()
    pltpu.sync_copy(acc_sh.at[pl.ds(sid * _NPT, _NPT)],
                    out_hbm.at[cid, pl.ds(sid * _NPT, _NPT)])


def _conv_sc(row_hbm, col_hbm, w_hbm, hwlo_hbm, hwhi_hbm, zero_hbm, out_hbm,
             ridx_v, cidx_v, w_v, rows_v, acc_sh, sem):
    cid = lax.axis_index("c")
    sid = lax.axis_index("s")
    pltpu.sync_copy(zero_hbm.at[pl.ds(sid * _NPT, _NPT)],
                    acc_sh.at[pl.ds(sid * _NPT, _NPT)])
    plsc.subcore_barrier()
    tile_base = sid * (_EK * _CONV_CHUNKS)

    def chunk_body(i, hw_src):
        base = pl.multiple_of(tile_base + i * _EK, 8)
        pltpu.sync_copy(row_hbm.at[pl.ds(base, _EK)], ridx_v)
        pltpu.sync_copy(col_hbm.at[pl.ds(base, _EK)], cidx_v)
        pltpu.sync_copy(w_hbm.at[pl.ds(base, _EK)], w_v)
        pltpu.async_copy(hw_src.at[ridx_v], rows_v, sem).wait()

        def mul_block(jb, _):
            wv = w_v[pl.ds(jb * 16, 16)]
            for l in range(16):
                iv = jnp.full((16,), l, jnp.int32)
                wj = wv.at[iv].get(mode="promise_in_bounds")
                j = jb * 16 + l
                for c in range(_HF // 16):
                    sl = (j, pl.ds(c * 16, 16))
                    rows_v[sl] = rows_v[sl] * wj
            return 0

        lax.fori_loop(0, _EK // 16, mul_block, 0)
        pltpu.sync_copy(rows_v, acc_sh.at[cidx_v], add=True)
        return 0

    @pl.when(cid == 0)
    def _():
        lax.fori_loop(0, _CONV_CHUNKS, lambda i, _: chunk_body(i, hwlo_hbm), 0)

    @pl.when(cid == 1)
    def _():
        lax.fori_loop(0, _CONV_CHUNKS, lambda i, _: chunk_body(i, hwhi_hbm), 0)

    plsc.subcore_barrier()
    pltpu.sync_copy(acc_sh.at[pl.ds(sid * _NPT, _NPT)],
                    out_hbm.at[cid, pl.ds(sid * _NPT, _NPT)])


def _mm(a, b):
    return lax.dot_general(a, b, (((1,), (1,)), ((), ())),
                           preferred_element_type=jnp.float32)


def _mmT(a, b):
    return lax.dot_general(a, b, (((0,), (0,)), ((), ())),
                           preferred_element_type=jnp.float32)


def _prep_tc(x_ref, linW_ref, linb_ref, W1_ref, degT_ref,
             dinv_ref, hwlo_ref, hwhi_ref):
    dinv = 1.0 / jnp.sqrt(degT_ref[:_N, 0:1] + degT_ref[:_N, 1:2] + 2.0)
    h0 = jnp.maximum(_mm(x_ref[...], linW_ref[...]) + linb_ref[...], 0.0)
    hw1 = _mm(h0, W1_ref[...]) * dinv
    dinv_ref[...] = dinv
    hwlo_ref[...] = hw1[:, :_HF]
    hwhi_ref[...] = hw1[:, _HF:]


_prep_call = pl.pallas_call(
    _prep_tc,
    out_shape=(
        jax.ShapeDtypeStruct((_N, 1), jnp.float32),
        jax.ShapeDtypeStruct((_N, _HF), jnp.float32),
        jax.ShapeDtypeStruct((_N, _HF), jnp.float32),
    ),
)


def _combine_body(acc2_ref, hwlo_ref, hwhi_ref, dinv_ref, b_ref, g_ref,
                  be_ref):
    dv = dinv_ref[...]
    pre_lo = acc2_ref[0, :_N, :] * dv + 2.0 * dv * hwlo_ref[...]
    pre_hi = acc2_ref[1, :_N, :] * dv + 2.0 * dv * hwhi_ref[...]
    pre = jnp.concatenate([pre_lo, pre_hi], axis=1) + b_ref[...]
    mu = jnp.mean(pre, axis=0, keepdims=True)
    var = jnp.mean((pre - mu) ** 2, axis=0, keepdims=True)
    return (pre - mu) / jnp.sqrt(var + _EPS) * g_ref[...] + be_ref[...]


def _mid_tc(acc2_ref, hwlo_ref, hwhi_ref, dinv_ref, b_ref, g_ref, be_ref,
            Wn_ref, hwlo_o, hwhi_o):
    h = _combine_body(acc2_ref, hwlo_ref, hwhi_ref, dinv_ref, b_ref, g_ref,
                      be_ref)
    hw = _mm(h, Wn_ref[...]) * dinv_ref[...]
    hwlo_o[...] = hw[:, :_HF]
    hwhi_o[...] = hw[:, _HF:]


_mid_call = pl.pallas_call(
    _mid_tc,
    out_shape=(
        jax.ShapeDtypeStruct((_N, _HF), jnp.float32),
        jax.ShapeDtypeStruct((_N, _HF), jnp.float32),
    ),
)


def _fin_tc(acc2_ref, hwlo_ref, hwhi_ref, dinv_ref, b_ref, g_ref, be_ref,
            h_o):
    h = _combine_body(acc2_ref, hwlo_ref, hwhi_ref, dinv_ref, b_ref, g_ref,
                      be_ref)
    h_o[...] = jnp.maximum(h, 0.0)


_fin_call = pl.pallas_call(
    _fin_tc,
    out_shape=jax.ShapeDtypeStruct((_N, _H), jnp.float32),
)


def _s2s_tc(h_ref, batch_ref, Wih_ref, Whh_ref, bih_ref, bhh_ref,
            mW0_ref, mW1_ref, mb0_ref, mb1_ref, o1W_ref, o1b_ref,
            o2W_ref, o2b_ref, out_ref):
    h = h_ref[...]
    onehot = (batch_ref[...] ==
              lax.broadcasted_iota(jnp.int32, (1, _B), 1)).astype(jnp.float32)
    q_star = jnp.zeros((_B, 2 * _H), jnp.float32)
    hh = jnp.zeros((_B, _H), jnp.float32)
    cc = jnp.zeros((_B, _H), jnp.float32)
    for _ in range(3):
        g = (_mm(q_star, Wih_ref[...]) + bih_ref[...] +
             _mm(hh, Whh_ref[...]) + bhh_ref[...])
        ii = g[:, 0 * _H:1 * _H]
        ff = g[:, 1 * _H:2 * _H]
        gg = g[:, 2 * _H:3 * _H]
        oo = g[:, 3 * _H:4 * _H]
        cc = jax.nn.sigmoid(ff) * cc + jax.nn.sigmoid(ii) * jnp.tanh(gg)
        hh = jax.nn.sigmoid(oo) * jnp.tanh(cc)
        q = hh
        hq = _mm(h, q)
        mg = jnp.max(hq, axis=0, keepdims=True)
        exm = jnp.exp(hq - mg) * onehot
        den = jnp.sum(exm, axis=0, keepdims=True)
        a = exm / (den + 1e-16)
        r = _mmT(a, h)
        q_star = jnp.concatenate([q, r], axis=1)
    z = jnp.maximum(_mm(q_star, mW0_ref[...]) + mb0_ref[...], 0.0)
    z = jnp.maximum(_mm(z, mW1_ref[...]) + mb1_ref[...], 0.0)
    z = jnp.maximum(_mm(z, o1W_ref[...]) + o1b_ref[...], 0.0)
    z = _mm(z, o2W_ref[...]) + o2b_ref[...]
    out_ref[...] = z


_s2s_call = pl.pallas_call(
    _s2s_tc,
    out_shape=jax.ShapeDtypeStruct((_B, 128), jnp.float32),
)


def kernel(x, edge_index, edge_dist, symmetry, global_idx, batch,
           lin_W, lin_b, conv_W, conv_b, bn_gamma, bn_beta,
           W_ih, W_hh, b_ih, b_hh, mlp_W, mlp_b,
           out1_W, out1_b, out2_W, out2_b):
    pad = _EPAD - _E
    rowp = jnp.concatenate([edge_index[0], jnp.zeros((pad,), jnp.int32)])
    colp = jnp.concatenate([edge_index[1], jnp.zeros((pad,), jnp.int32)])
    wp = jnp.concatenate([edge_dist, jnp.zeros((pad,), jnp.float32)])
    zeros1 = jnp.zeros((_NPAD,), jnp.float32)
    zeros2 = jnp.zeros((_NPAD, _HF), jnp.float32)

    deg2 = _deg_call()(colp, wp, zeros1)
    degT = deg2.T
    dinv, hwlo, hwhi = _prep_call(x, lin_W, lin_b.reshape(1, _H),
                                  conv_W[0], degT)
    h_fin = None
    for i in range(_NCONV):
        acc2 = _conv_call()(rowp, colp, wp, hwlo, hwhi, zeros2)
        args = (acc2, hwlo, hwhi, dinv, conv_b[i].reshape(1, _H),
                bn_gamma.reshape(1, _H), bn_beta.reshape(1, _H))
        if i < _NCONV - 1:
            hwlo, hwhi = _mid_call(*args, conv_W[i + 1])
        else:
            h_fin = _fin_call(*args)

    o2W_pad = jnp.zeros((128, 32), jnp.float32).at[0:1].set(out2_W)
    o2b_pad = jnp.zeros((1, 128), jnp.float32).at[0, 0].set(out2_b[0])
    z = _s2s_call(h_fin, batch.reshape(_N, 1),
                  W_ih, W_hh, b_ih.reshape(1, 4 * _H), b_hh.reshape(1, 4 * _H),
                  mlp_W[0], mlp_W[1], mlp_b[0].reshape(1, 2 * _H),
                  mlp_b[1].reshape(1, 2 * _H), out1_W,
                  out1_b.reshape(1, 32), o2W_pad, o2b_pad)
    return z[:, 0].reshape(1, _B)


# speedup vs baseline: 7.0783x; 1.5960x over previous
import functools

import jax
import jax.numpy as jnp
from jax import lax
from jax.experimental import pallas as pl
from jax.experimental.pallas import tpu as pltpu
from jax.experimental.pallas import tpu_sc as plsc

_N = 10000
_E = 160000
_H = 256
_HF = 128
_B = 16
_EPS = 1e-5
_NCONV = 5

_NC = 2
_NS = 16
_NPAD = 10240
_NPT = _NPAD // _NS
_EK = 128
_CONV_CHUNKS = 80
_PH = 2
_PCH = _CONV_CHUNKS // _PH
_EPAD = _NS * _EK * _CONV_CHUNKS
_DEG_CHUNKS = _EPAD // (_NC * _NS * _EK)


def _sc_mesh():
    return plsc.VectorSubcoreMesh(core_axis_name="c", subcore_axis_name="s",
                                  num_cores=_NC, num_subcores=_NS)


_SC_CALLS = {}


def _deg_call():
    if "deg" not in _SC_CALLS:
        _SC_CALLS["deg"] = pl.kernel(
            _deg_sc,
            out_type=jax.ShapeDtypeStruct((_NC, _NPAD), jnp.float32),
            mesh=_sc_mesh(),
            scratch_types=[
                pltpu.VMEM((_DEG_CHUNKS, _EK), jnp.int32),
                pltpu.VMEM((_DEG_CHUNKS, _EK), jnp.float32),
                pltpu.VMEM_SHARED((_NPAD,), jnp.float32),
            ],
        )
    return _SC_CALLS["deg"]


def _conv_call():
    if "conv" not in _SC_CALLS:
        _SC_CALLS["conv"] = pl.kernel(
            _conv_sc,
            out_type=jax.ShapeDtypeStruct((_NC, _NPAD, _HF), jnp.float32),
            mesh=_sc_mesh(),
            scratch_types=[
                pltpu.VMEM((_PCH, _EK), jnp.int32),
                pltpu.VMEM((_PCH, _EK), jnp.int32),
                pltpu.VMEM((_PCH, _EK), jnp.float32),
                pltpu.VMEM((_EK, _HF), jnp.float32),
                pltpu.VMEM((_EK, _HF), jnp.float32),
                pltpu.VMEM_SHARED((_NPAD, _HF), jnp.float32),
                pltpu.SemaphoreType.DMA,
                pltpu.SemaphoreType.DMA,
            ],
        )
    return _SC_CALLS["conv"]


def _deg_sc(col_hbm, w_hbm, zero_hbm, out_hbm, cidx_all, w_all, acc_sh):
    cid = lax.axis_index("c")
    sid = lax.axis_index("s")
    wid = cid * _NS + sid
    pltpu.sync_copy(zero_hbm.at[pl.ds(sid * _NPT, _NPT)],
                    acc_sh.at[pl.ds(sid * _NPT, _NPT)])
    pltpu.sync_copy(col_hbm.at[wid], cidx_all)
    pltpu.sync_copy(w_hbm.at[wid], w_all)
    plsc.subcore_barrier()

    def chunk(i, _):
        pltpu.sync_copy(w_all.at[i], acc_sh.at[cidx_all.at[i]], add=True)
        return 0

    lax.fori_loop(0, _DEG_CHUNKS, chunk, 0)
    plsc.subcore_barrier()
    pltpu.sync_copy(acc_sh.at[pl.ds(sid * _NPT, _NPT)],
                    out_hbm.at[cid, pl.ds(sid * _NPT, _NPT)])


def _conv_sc(row_hbm, col_hbm, w_hbm, hwlo_hbm, hwhi_hbm, zero_hbm, out_hbm,
             ridx_all, cidx_all, w_all, buf_a, buf_b, acc_sh, sem_a, sem_b):
    cid = lax.axis_index("c")
    sid = lax.axis_index("s")
    pltpu.sync_copy(zero_hbm.at[pl.ds(sid * _NPT, _NPT)],
                    acc_sh.at[pl.ds(sid * _NPT, _NPT)])
    plsc.subcore_barrier()

    def work(hw_src):
        def gather_start(ch, buf, sem):
            pltpu.async_copy(hw_src.at[ridx_all.at[ch]], buf, sem)

        def gather_wait(buf, sem):
            pltpu.make_async_copy(hw_src.at[ridx_all.at[0]], buf, sem).wait()

        def process(ch, buf):
            def mul_block(jb, _):
                wv = w_all[ch, pl.ds(jb * 16, 16)]
                for l in range(16):
                    iv = jnp.full((16,), l, jnp.int32)
                    wj = wv.at[iv].get(mode="promise_in_bounds")
                    j = jb * 16 + l
                    for c in range(_HF // 16):
                        sl = (j, pl.ds(c * 16, 16))
                        buf[sl] = buf[sl] * wj
                return 0

            lax.fori_loop(0, _EK // 16, mul_block, 0)
            pltpu.sync_copy(buf, acc_sh.at[cidx_all.at[ch]], add=True)

        def phase(ph, _):
            pltpu.sync_copy(row_hbm.at[sid, ph], ridx_all)
            pltpu.sync_copy(col_hbm.at[sid, ph], cidx_all)
            pltpu.sync_copy(w_hbm.at[sid, ph], w_all)
            gather_start(0, buf_a, sem_a)

            def pair(t, _):
                ch0 = 2 * t
                gather_start(ch0 + 1, buf_b, sem_b)
                gather_wait(buf_a, sem_a)
                process(ch0, buf_a)

                @pl.when(t < _PCH // 2 - 1)
                def _():
                    gather_start(ch0 + 2, buf_a, sem_a)

                gather_wait(buf_b, sem_b)
                process(ch0 + 1, buf_b)
                return 0

            lax.fori_loop(0, _PCH // 2, pair, 0)
            return 0

        lax.fori_loop(0, _PH, phase, 0)

    @pl.when(cid == 0)
    def _():
        work(hwlo_hbm)

    @pl.when(cid == 1)
    def _():
        work(hwhi_hbm)

    plsc.subcore_barrier()
    pltpu.sync_copy(acc_sh.at[pl.ds(sid * _NPT, _NPT)],
                    out_hbm.at[cid, pl.ds(sid * _NPT, _NPT)])


def _mm(a, b):
    return lax.dot_general(a, b, (((1,), (1,)), ((), ())),
                           preferred_element_type=jnp.float32)


def _mmT(a, b):
    return lax.dot_general(a, b, (((0,), (0,)), ((), ())),
                           preferred_element_type=jnp.float32)


def _prep_tc(x_ref, linW_ref, linb_ref, W1_ref, degT_ref,
             dinv_ref, hwlo_ref, hwhi_ref):
    dinv = 1.0 / jnp.sqrt(degT_ref[:_N, 0:1] + degT_ref[:_N, 1:2] + 2.0)
    h0 = jnp.maximum(_mm(x_ref[...], linW_ref[...]) + linb_ref[...], 0.0)
    hw1 = _mm(h0, W1_ref[...]) * dinv
    dinv_ref[...] = dinv
    hwlo_ref[...] = hw1[:, :_HF]
    hwhi_ref[...] = hw1[:, _HF:]


_prep_call = pl.pallas_call(
    _prep_tc,
    out_shape=(
        jax.ShapeDtypeStruct((_N, 1), jnp.float32),
        jax.ShapeDtypeStruct((_N, _HF), jnp.float32),
        jax.ShapeDtypeStruct((_N, _HF), jnp.float32),
    ),
)


def _combine_body(acc2_ref, hwlo_ref, hwhi_ref, dinv_ref, b_ref, g_ref,
                  be_ref):
    dv = dinv_ref[...]
    pre_lo = acc2_ref[0, :_N, :] * dv + 2.0 * dv * hwlo_ref[...]
    pre_hi = acc2_ref[1, :_N, :] * dv + 2.0 * dv * hwhi_ref[...]
    pre = jnp.concatenate([pre_lo, pre_hi], axis=1) + b_ref[...]
    mu = jnp.mean(pre, axis=0, keepdims=True)
    var = jnp.mean((pre - mu) ** 2, axis=0, keepdims=True)
    return (pre - mu) / jnp.sqrt(var + _EPS) * g_ref[...] + be_ref[...]


def _mid_tc(acc2_ref, hwlo_ref, hwhi_ref, dinv_ref, b_ref, g_ref, be_ref,
            Wn_ref, hwlo_o, hwhi_o):
    h = _combine_body(acc2_ref, hwlo_ref, hwhi_ref, dinv_ref, b_ref, g_ref,
                      be_ref)
    hw = _mm(h, Wn_ref[...]) * dinv_ref[...]
    hwlo_o[...] = hw[:, :_HF]
    hwhi_o[...] = hw[:, _HF:]


_mid_call = pl.pallas_call(
    _mid_tc,
    out_shape=(
        jax.ShapeDtypeStruct((_N, _HF), jnp.float32),
        jax.ShapeDtypeStruct((_N, _HF), jnp.float32),
    ),
)


def _fin_tc(acc2_ref, hwlo_ref, hwhi_ref, dinv_ref, b_ref, g_ref, be_ref,
            h_o):
    h = _combine_body(acc2_ref, hwlo_ref, hwhi_ref, dinv_ref, b_ref, g_ref,
                      be_ref)
    h_o[...] = jnp.maximum(h, 0.0)


_fin_call = pl.pallas_call(
    _fin_tc,
    out_shape=jax.ShapeDtypeStruct((_N, _H), jnp.float32),
)


def _s2s_tc(h_ref, batch_ref, Wih_ref, Whh_ref, bih_ref, bhh_ref,
            mW0_ref, mW1_ref, mb0_ref, mb1_ref, o1W_ref, o1b_ref,
            o2W_ref, o2b_ref, out_ref):
    h = h_ref[...]
    onehot = (batch_ref[...] ==
              lax.broadcasted_iota(jnp.int32, (1, _B), 1)).astype(jnp.float32)
    q_star = jnp.zeros((_B, 2 * _H), jnp.float32)
    hh = jnp.zeros((_B, _H), jnp.float32)
    cc = jnp.zeros((_B, _H), jnp.float32)
    for _ in range(3):
        g = (_mm(q_star, Wih_ref[...]) + bih_ref[...] +
             _mm(hh, Whh_ref[...]) + bhh_ref[...])
        ii = g[:, 0 * _H:1 * _H]
        ff = g[:, 1 * _H:2 * _H]
        gg = g[:, 2 * _H:3 * _H]
        oo = g[:, 3 * _H:4 * _H]
        cc = jax.nn.sigmoid(ff) * cc + jax.nn.sigmoid(ii) * jnp.tanh(gg)
        hh = jax.nn.sigmoid(oo) * jnp.tanh(cc)
        q = hh
        hq = _mm(h, q)
        mg = jnp.max(hq, axis=0, keepdims=True)
        exm = jnp.exp(hq - mg) * onehot
        den = jnp.sum(exm, axis=0, keepdims=True)
        a = exm / (den + 1e-16)
        r = _mmT(a, h)
        q_star = jnp.concatenate([q, r], axis=1)
    z = jnp.maximum(_mm(q_star, mW0_ref[...]) + mb0_ref[...], 0.0)
    z = jnp.maximum(_mm(z, mW1_ref[...]) + mb1_ref[...], 0.0)
    z = jnp.maximum(_mm(z, o1W_ref[...]) + o1b_ref[...], 0.0)
    z = _mm(z, o2W_ref[...]) + o2b_ref[...]
    out_ref[...] = z


_s2s_call = pl.pallas_call(
    _s2s_tc,
    out_shape=jax.ShapeDtypeStruct((_B, 128), jnp.float32),
)


def kernel(x, edge_index, edge_dist, symmetry, global_idx, batch,
           lin_W, lin_b, conv_W, conv_b, bn_gamma, bn_beta,
           W_ih, W_hh, b_ih, b_hh, mlp_W, mlp_b,
           out1_W, out1_b, out2_W, out2_b):
    pad = _EPAD - _E
    rowp = jnp.concatenate([edge_index[0], jnp.zeros((pad,), jnp.int32)])
    colp = jnp.concatenate([edge_index[1], jnp.zeros((pad,), jnp.int32)])
    wp = jnp.concatenate([edge_dist, jnp.zeros((pad,), jnp.float32)])
    zeros1 = jnp.zeros((_NPAD,), jnp.float32)
    zeros2 = jnp.zeros((_NPAD, _HF), jnp.float32)

    row3 = rowp.reshape(_NS, _PH, _PCH, _EK)
    col3 = colp.reshape(_NS, _PH, _PCH, _EK)
    w3 = wp.reshape(_NS, _PH, _PCH, _EK)
    col32 = colp.reshape(_NC * _NS, _DEG_CHUNKS, _EK)
    w32 = wp.reshape(_NC * _NS, _DEG_CHUNKS, _EK)

    deg2 = _deg_call()(col32, w32, zeros1)
    degT = deg2.T
    dinv, hwlo, hwhi = _prep_call(x, lin_W, lin_b.reshape(1, _H),
                                  conv_W[0], degT)
    h_fin = None
    for i in range(_NCONV):
        acc2 = _conv_call()(row3, col3, w3, hwlo, hwhi, zeros2)
        args = (acc2, hwlo, hwhi, dinv, conv_b[i].reshape(1, _H),
                bn_gamma.reshape(1, _H), bn_beta.reshape(1, _H))
        if i < _NCONV - 1:
            hwlo, hwhi = _mid_call(*args, conv_W[i + 1])
        else:
            h_fin = _fin_call(*args)

    o2W_pad = jnp.zeros((128, 32), jnp.float32).at[0:1].set(out2_W)
    o2b_pad = jnp.zeros((1, 128), jnp.float32).at[0, 0].set(out2_b[0])
    z = _s2s_call(h_fin, batch.reshape(_N, 1),
                  W_ih, W_hh, b_ih.reshape(1, 4 * _H), b_hh.reshape(1, 4 * _H),
                  mlp_W[0], mlp_W[1], mlp_b[0].reshape(1, 2 * _H),
                  mlp_b[1].reshape(1, 2 * _H), out1_W,
                  out1_b.reshape(1, 32), o2W_pad, o2b_pad)
    return z[:, 0].reshape(1, _B)


# speedup vs baseline: 8.3298x; 1.1768x over previous
import functools

import jax
import jax.numpy as jnp
from jax import lax
from jax.experimental import pallas as pl
from jax.experimental.pallas import tpu as pltpu
from jax.experimental.pallas import tpu_sc as plsc

_N = 10000
_E = 160000
_H = 256
_HF = 128
_B = 16
_EPS = 1e-5
_NCONV = 5

_NC = 2
_NS = 16
_NPAD = 10240
_NPT = _NPAD // _NS
_EK = 128
_CONV_CHUNKS = 80
_PH = 2
_PCH = _CONV_CHUNKS // _PH
_EPAD = _NS * _EK * _CONV_CHUNKS
_DEG_CHUNKS = _EPAD // (_NC * _NS * _EK)


def _sc_mesh():
    return plsc.VectorSubcoreMesh(core_axis_name="c", subcore_axis_name="s",
                                  num_cores=_NC, num_subcores=_NS)


_SC_CALLS = {}


def _deg_call():
    if "deg" not in _SC_CALLS:
        _SC_CALLS["deg"] = pl.kernel(
            _deg_sc,
            out_type=jax.ShapeDtypeStruct((_NC, _NPAD), jnp.float32),
            mesh=_sc_mesh(),
            scratch_types=[
                pltpu.VMEM((_DEG_CHUNKS, _EK), jnp.int32),
                pltpu.VMEM((_DEG_CHUNKS, _EK), jnp.float32),
                pltpu.VMEM_SHARED((_NPAD,), jnp.float32),
            ],
        )
    return _SC_CALLS["deg"]


def _conv_call():
    if "conv" not in _SC_CALLS:
        _SC_CALLS["conv"] = pl.kernel(
            _conv_sc,
            out_type=jax.ShapeDtypeStruct((_NC, _NPAD, _HF), jnp.float32),
            mesh=_sc_mesh(),
            scratch_types=[
                pltpu.VMEM((_PCH, _EK), jnp.int32),
                pltpu.VMEM((_PCH, _EK), jnp.int32),
                pltpu.VMEM((_PCH, _EK), jnp.float32),
                pltpu.VMEM((_EK, _HF), jnp.float32),
                pltpu.VMEM((_EK, _HF), jnp.float32),
                pltpu.VMEM_SHARED((_NPAD, _HF), jnp.float32),
                pltpu.SemaphoreType.DMA,
                pltpu.SemaphoreType.DMA,
                pltpu.SemaphoreType.DMA,
                pltpu.SemaphoreType.DMA,
            ],
        )
    return _SC_CALLS["conv"]


def _deg_sc(col_hbm, w_hbm, zero_hbm, out_hbm, cidx_all, w_all, acc_sh):
    cid = lax.axis_index("c")
    sid = lax.axis_index("s")
    wid = cid * _NS + sid
    pltpu.sync_copy(zero_hbm.at[pl.ds(sid * _NPT, _NPT)],
                    acc_sh.at[pl.ds(sid * _NPT, _NPT)])
    pltpu.sync_copy(col_hbm.at[wid], cidx_all)
    pltpu.sync_copy(w_hbm.at[wid], w_all)
    plsc.subcore_barrier()

    def chunk(i, _):
        pltpu.sync_copy(w_all.at[i], acc_sh.at[cidx_all.at[i]], add=True)
        return 0

    lax.fori_loop(0, _DEG_CHUNKS, chunk, 0)
    plsc.subcore_barrier()
    pltpu.sync_copy(acc_sh.at[pl.ds(sid * _NPT, _NPT)],
                    out_hbm.at[cid, pl.ds(sid * _NPT, _NPT)])


def _conv_sc(row_hbm, col_hbm, w_hbm, hwlo_hbm, hwhi_hbm, zero_hbm, out_hbm,
             ridx_all, cidx_all, w_all, buf_a, buf_b, acc_sh,
             sem_a, sem_b, sem_sa, sem_sb):
    cid = lax.axis_index("c")
    sid = lax.axis_index("s")
    pltpu.sync_copy(zero_hbm.at[pl.ds(sid * _NPT, _NPT)],
                    acc_sh.at[pl.ds(sid * _NPT, _NPT)])
    plsc.subcore_barrier()

    def work(hw_src):
        def gather_start(ch, buf, sem):
            pltpu.async_copy(hw_src.at[ridx_all.at[ch]], buf, sem)

        def gather_wait(buf, sem):
            pltpu.make_async_copy(hw_src.at[ridx_all.at[0]], buf, sem).wait()

        def mul(ch, buf):
            def mul_block(jb, _):
                wv = w_all[ch, pl.ds(jb * 16, 16)]
                for l in range(16):
                    iv = jnp.full((16,), l, jnp.int32)
                    wj = wv.at[iv].get(mode="promise_in_bounds")
                    j = jb * 16 + l
                    for c in range(_HF // 16):
                        sl = (j, pl.ds(c * 16, 16))
                        buf[sl] = buf[sl] * wj
                return 0

            lax.fori_loop(0, _EK // 16, mul_block, 0)

        def scat_start(ch, buf, sem):
            pltpu.async_copy(buf, acc_sh.at[cidx_all.at[ch]], sem, add=True)

        def scat_wait(buf, sem):
            pltpu.make_async_copy(buf, acc_sh.at[cidx_all.at[0]], sem).wait()

        def phase(ph, _):
            pltpu.sync_copy(row_hbm.at[sid, ph], ridx_all)
            pltpu.sync_copy(col_hbm.at[sid, ph], cidx_all)
            pltpu.sync_copy(w_hbm.at[sid, ph], w_all)
            gather_start(0, buf_a, sem_a)
            gather_start(1, buf_b, sem_b)

            def pair(t, _):
                ch0 = 2 * t
                gather_wait(buf_a, sem_a)
                mul(ch0, buf_a)
                scat_start(ch0, buf_a, sem_sa)
                gather_wait(buf_b, sem_b)
                mul(ch0 + 1, buf_b)
                scat_start(ch0 + 1, buf_b, sem_sb)
                scat_wait(buf_a, sem_sa)

                @pl.when(t < _PCH // 2 - 1)
                def _():
                    gather_start(ch0 + 2, buf_a, sem_a)

                scat_wait(buf_b, sem_sb)

                @pl.when(t < _PCH // 2 - 1)
                def _():
                    gather_start(ch0 + 3, buf_b, sem_b)

                return 0

            lax.fori_loop(0, _PCH // 2, pair, 0)
            return 0

        lax.fori_loop(0, _PH, phase, 0)

    @pl.when(cid == 0)
    def _():
        work(hwlo_hbm)

    @pl.when(cid == 1)
    def _():
        work(hwhi_hbm)

    plsc.subcore_barrier()
    pltpu.sync_copy(acc_sh.at[pl.ds(sid * _NPT, _NPT)],
                    out_hbm.at[cid, pl.ds(sid * _NPT, _NPT)])


def _mm(a, b):
    return lax.dot_general(a, b, (((1,), (1,)), ((), ())),
                           preferred_element_type=jnp.float32)


def _mmT(a, b):
    return lax.dot_general(a, b, (((0,), (0,)), ((), ())),
                           preferred_element_type=jnp.float32)


def _prep_tc(x_ref, linW_ref, linb_ref, W1_ref, degT_ref,
             dinv_ref, hwlo_ref, hwhi_ref):
    dinv = 1.0 / jnp.sqrt(degT_ref[:_N, 0:1] + degT_ref[:_N, 1:2] + 2.0)
    h0 = jnp.maximum(_mm(x_ref[...], linW_ref[...]) + linb_ref[...], 0.0)
    hw1 = _mm(h0, W1_ref[...]) * dinv
    dinv_ref[...] = dinv
    hwlo_ref[...] = hw1[:, :_HF]
    hwhi_ref[...] = hw1[:, _HF:]


_prep_call = pl.pallas_call(
    _prep_tc,
    out_shape=(
        jax.ShapeDtypeStruct((_N, 1), jnp.float32),
        jax.ShapeDtypeStruct((_N, _HF), jnp.float32),
        jax.ShapeDtypeStruct((_N, _HF), jnp.float32),
    ),
)


def _combine_body(acc2_ref, hwlo_ref, hwhi_ref, dinv_ref, b_ref, g_ref,
                  be_ref):
    dv = dinv_ref[...]
    pre_lo = acc2_ref[0, :_N, :] * dv + 2.0 * dv * hwlo_ref[...]
    pre_hi = acc2_ref[1, :_N, :] * dv + 2.0 * dv * hwhi_ref[...]
    pre = jnp.concatenate([pre_lo, pre_hi], axis=1) + b_ref[...]
    mu = jnp.mean(pre, axis=0, keepdims=True)
    var = jnp.mean((pre - mu) ** 2, axis=0, keepdims=True)
    return (pre - mu) / jnp.sqrt(var + _EPS) * g_ref[...] + be_ref[...]


def _mid_tc(acc2_ref, hwlo_ref, hwhi_ref, dinv_ref, b_ref, g_ref, be_ref,
            Wn_ref, hwlo_o, hwhi_o):
    h = _combine_body(acc2_ref, hwlo_ref, hwhi_ref, dinv_ref, b_ref, g_ref,
                      be_ref)
    hw = _mm(h, Wn_ref[...]) * dinv_ref[...]
    hwlo_o[...] = hw[:, :_HF]
    hwhi_o[...] = hw[:, _HF:]


_mid_call = pl.pallas_call(
    _mid_tc,
    out_shape=(
        jax.ShapeDtypeStruct((_N, _HF), jnp.float32),
        jax.ShapeDtypeStruct((_N, _HF), jnp.float32),
    ),
)


def _fin_tc(acc2_ref, hwlo_ref, hwhi_ref, dinv_ref, b_ref, g_ref, be_ref,
            h_o):
    h = _combine_body(acc2_ref, hwlo_ref, hwhi_ref, dinv_ref, b_ref, g_ref,
                      be_ref)
    h_o[...] = jnp.maximum(h, 0.0)


_fin_call = pl.pallas_call(
    _fin_tc,
    out_shape=jax.ShapeDtypeStruct((_N, _H), jnp.float32),
)


def _s2s_tc(h_ref, batch_ref, Wih_ref, Whh_ref, bih_ref, bhh_ref,
            mW0_ref, mW1_ref, mb0_ref, mb1_ref, o1W_ref, o1b_ref,
            o2W_ref, o2b_ref, out_ref):
    h = h_ref[...]
    onehot = (batch_ref[...] ==
              lax.broadcasted_iota(jnp.int32, (1, _B), 1)).astype(jnp.float32)
    q_star = jnp.zeros((_B, 2 * _H), jnp.float32)
    hh = jnp.zeros((_B, _H), jnp.float32)
    cc = jnp.zeros((_B, _H), jnp.float32)
    for _ in range(3):
        g = (_mm(q_star, Wih_ref[...]) + bih_ref[...] +
             _mm(hh, Whh_ref[...]) + bhh_ref[...])
        ii = g[:, 0 * _H:1 * _H]
        ff = g[:, 1 * _H:2 * _H]
        gg = g[:, 2 * _H:3 * _H]
        oo = g[:, 3 * _H:4 * _H]
        cc = jax.nn.sigmoid(ff) * cc + jax.nn.sigmoid(ii) * jnp.tanh(gg)
        hh = jax.nn.sigmoid(oo) * jnp.tanh(cc)
        q = hh
        hq = _mm(h, q)
        mg = jnp.max(hq, axis=0, keepdims=True)
        exm = jnp.exp(hq - mg) * onehot
        den = jnp.sum(exm, axis=0, keepdims=True)
        a = exm / (den + 1e-16)
        r = _mmT(a, h)
        q_star = jnp.concatenate([q, r], axis=1)
    z = jnp.maximum(_mm(q_star, mW0_ref[...]) + mb0_ref[...], 0.0)
    z = jnp.maximum(_mm(z, mW1_ref[...]) + mb1_ref[...], 0.0)
    z = jnp.maximum(_mm(z, o1W_ref[...]) + o1b_ref[...], 0.0)
    z = _mm(z, o2W_ref[...]) + o2b_ref[...]
    out_ref[...] = z


_s2s_call = pl.pallas_call(
    _s2s_tc,
    out_shape=jax.ShapeDtypeStruct((_B, 128), jnp.float32),
)


def kernel(x, edge_index, edge_dist, symmetry, global_idx, batch,
           lin_W, lin_b, conv_W, conv_b, bn_gamma, bn_beta,
           W_ih, W_hh, b_ih, b_hh, mlp_W, mlp_b,
           out1_W, out1_b, out2_W, out2_b):
    pad = _EPAD - _E
    rowp = jnp.concatenate([edge_index[0], jnp.zeros((pad,), jnp.int32)])
    colp = jnp.concatenate([edge_index[1], jnp.zeros((pad,), jnp.int32)])
    wp = jnp.concatenate([edge_dist, jnp.zeros((pad,), jnp.float32)])
    zeros1 = jnp.zeros((_NPAD,), jnp.float32)
    zeros2 = jnp.zeros((_NPAD, _HF), jnp.float32)

    row3 = rowp.reshape(_NS, _PH, _PCH, _EK)
    col3 = colp.reshape(_NS, _PH, _PCH, _EK)
    w3 = wp.reshape(_NS, _PH, _PCH, _EK)
    col32 = colp.reshape(_NC * _NS, _DEG_CHUNKS, _EK)
    w32 = wp.reshape(_NC * _NS, _DEG_CHUNKS, _EK)

    deg2 = _deg_call()(col32, w32, zeros1)
    degT = deg2.T
    dinv, hwlo, hwhi = _prep_call(x, lin_W, lin_b.reshape(1, _H),
                                  conv_W[0], degT)
    h_fin = None
    for i in range(_NCONV):
        acc2 = _conv_call()(row3, col3, w3, hwlo, hwhi, zeros2)
        args = (acc2, hwlo, hwhi, dinv, conv_b[i].reshape(1, _H),
                bn_gamma.reshape(1, _H), bn_beta.reshape(1, _H))
        if i < _NCONV - 1:
            hwlo, hwhi = _mid_call(*args, conv_W[i + 1])
        else:
            h_fin = _fin_call(*args)

    o2W_pad = jnp.zeros((128, 32), jnp.float32).at[0:1].set(out2_W)
    o2b_pad = jnp.zeros((1, 128), jnp.float32).at[0, 0].set(out2_b[0])
    z = _s2s_call(h_fin, batch.reshape(_N, 1),
                  W_ih, W_hh, b_ih.reshape(1, 4 * _H), b_hh.reshape(1, 4 * _H),
                  mlp_W[0], mlp_W[1], mlp_b[0].reshape(1, 2 * _H),
                  mlp_b[1].reshape(1, 2 * _H), out1_W,
                  out1_b.reshape(1, 32), o2W_pad, o2b_pad)
    return z[:, 0].reshape(1, _B)
